# Initial kernel scaffold; baseline (speedup 1.0000x reference)
import functools

import jax
import jax.numpy as jnp
from jax import lax
from jax.experimental import pallas as pl
from jax.experimental.pallas import tpu as pltpu
from jax.experimental.pallas import tpu_sc as plsc

_N = 10000
_E = 320000
_D = 128
_NC = 2
_NS = 16
_NW = _NC * _NS
_EPW = _E // _NW
_CH = 125
_NCHUNK = _EPW // _CH
_RPT = _N // _NS
_ZR = 125
_NZ = _RPT // _ZR

_mesh = plsc.VectorSubcoreMesh(core_axis_name="c", subcore_axis_name="s")


@functools.partial(
    pl.kernel,
    out_type=jax.ShapeDtypeStruct((_NC, 2, _N, 16), jnp.float32),
    mesh=_mesh,
    scratch_types=[
        pltpu.VMEM((_NCHUNK, _CH), jnp.int32),
        pltpu.VMEM((_NCHUNK, _CH), jnp.int32),
        pltpu.VMEM((_CH, 16), jnp.float32),
        pltpu.VMEM((_RPT, 16), jnp.float32),
        pltpu.VMEM_SHARED((_N, 16), jnp.float32),
        pltpu.VMEM_SHARED((_N, 16), jnp.float32),
    ],
)
def _sc_degrees(srcg, dstg, out, src_v, dst_v, ones_v, zero_v, dego, degi):
    cid = lax.axis_index("c")
    sid = lax.axis_index("s")
    wid = sid * _NC + cid
    pltpu.sync_copy(srcg.at[wid], src_v)
    pltpu.sync_copy(dstg.at[wid], dst_v)
    one16 = jnp.ones((16,), jnp.float32)
    zero16 = jnp.zeros((16,), jnp.float32)

    def fill_ones(j, c):
        ones_v[j] = one16
        return c

    lax.fori_loop(0, _CH, fill_ones, 0)

    def fill_zero(j, c):
        zero_v[j] = zero16
        return c

    lax.fori_loop(0, _RPT, fill_zero, 0)

    base = sid * _RPT
    pltpu.sync_copy(zero_v, dego.at[pl.ds(base, _RPT)])
    pltpu.sync_copy(zero_v, degi.at[pl.ds(base, _RPT)])
    plsc.subcore_barrier()

    def body(j, c):
        pltpu.sync_copy(ones_v, dego.at[src_v.at[j]], add=True)
        pltpu.sync_copy(ones_v, degi.at[dst_v.at[j]], add=True)
        return c

    lax.fori_loop(0, _NCHUNK, body, 0)
    plsc.subcore_barrier()
    pltpu.sync_copy(dego.at[pl.ds(base, _RPT)], out.at[cid, 0, pl.ds(base, _RPT)])
    pltpu.sync_copy(degi.at[pl.ds(base, _RPT)], out.at[cid, 1, pl.ds(base, _RPT)])


@functools.partial(
    pl.kernel,
    out_type=jax.ShapeDtypeStruct((_NC, _N, _D), jnp.float32),
    mesh=_mesh,
    scratch_types=[
        pltpu.VMEM((_NCHUNK, _CH), jnp.int32),
        pltpu.VMEM((_NCHUNK, _CH), jnp.int32),
        pltpu.VMEM((_CH, _D), jnp.float32),
        pltpu.VMEM_SHARED((_N, _D), jnp.float32),
        pltpu.SemaphoreType.DMA,
    ],
)
def _sc_gather_scatter(h, srcg, dstg, out, src_v, dst_v, rows, accum, sem):
    cid = lax.axis_index("c")
    sid = lax.axis_index("s")
    wid = sid * _NC + cid
    pltpu.sync_copy(srcg.at[wid], src_v)
    pltpu.sync_copy(dstg.at[wid], dst_v)
    zero16 = jnp.zeros((16,), jnp.float32)

    def fill_zero(j, c):
        for k in range(_D // 16):
            rows[j, pl.ds(k * 16, 16)] = zero16
        return c

    lax.fori_loop(0, _CH, fill_zero, 0)
    base = sid * _RPT
    for t in range(_NZ):
        pltpu.sync_copy(rows, accum.at[pl.ds(base + t * _ZR, _ZR)])
    plsc.subcore_barrier()

    def body(j, c):
        pltpu.async_copy(h.at[src_v.at[j]], rows, sem).wait()
        pltpu.sync_copy(rows, accum.at[dst_v.at[j]], add=True)
        return c

    lax.fori_loop(0, _NCHUNK, body, 0)
    plsc.subcore_barrier()
    for t in range(_NZ):
        off = base + t * _ZR
        pltpu.sync_copy(accum.at[pl.ds(off, _ZR)], out.at[cid, pl.ds(off, _ZR)])


_R = 2000
_G = _N // _R


def _tc_degprep_body(dp_ref, ro_ref, ri_ref):
    dego = jnp.sum(dp_ref[0, 0] + dp_ref[1, 0], axis=1) * (1.0 / 16.0)
    degi = jnp.sum(dp_ref[0, 1] + dp_ref[1, 1], axis=1) * (1.0 / 16.0)
    ro = lax.rsqrt(jnp.maximum(dego, 1.0))
    ri = lax.rsqrt(jnp.maximum(degi, 1.0))
    ro_ref[...] = jnp.broadcast_to(ro[:, None], (_R, _D))
    ri_ref[...] = jnp.broadcast_to(ri[:, None], (_R, _D))


def _degprep(dp):
    return pl.pallas_call(
        _tc_degprep_body,
        grid=(_G,),
        in_specs=[pl.BlockSpec((_NC, 2, _R, 16), lambda i: (0, 0, i, 0))],
        out_specs=[pl.BlockSpec((_R, _D), lambda i: (i, 0))] * 2,
        out_shape=[jax.ShapeDtypeStruct((_N, _D), jnp.float32)] * 2,
    )(dp)


def _tc_prep_body(x_ref, w_ref, ro_ref, h_ref):
    h_ref[...] = (
        jnp.dot(x_ref[...], w_ref[...], preferred_element_type=jnp.float32)
        * ro_ref[...]
    )


def _prep(x, w, ro):
    return pl.pallas_call(
        _tc_prep_body,
        grid=(_G,),
        in_specs=[
            pl.BlockSpec((_R, _D), lambda i: (i, 0)),
            pl.BlockSpec((_D, _D), lambda i: (0, 0)),
            pl.BlockSpec((_R, _D), lambda i: (i, 0)),
        ],
        out_specs=pl.BlockSpec((_R, _D), lambda i: (i, 0)),
        out_shape=jax.ShapeDtypeStruct((_N, _D), jnp.float32),
    )(x, w, ro)


def _tc_step_body(s_ref, ri_ref, b_ref, w_ref, ro_ref, y_ref, h_ref):
    agg = (s_ref[0] + s_ref[1]) * ri_ref[...] + b_ref[...]
    y = jnp.maximum(agg, 0.0)
    y_ref[...] = y
    h_ref[...] = (
        jnp.dot(y, w_ref[...], preferred_element_type=jnp.float32) * ro_ref[...]
    )


def _step(s, ri, b, w, ro):
    return pl.pallas_call(
        _tc_step_body,
        grid=(_G,),
        in_specs=[
            pl.BlockSpec((_NC, _R, _D), lambda i: (0, i, 0)),
            pl.BlockSpec((_R, _D), lambda i: (i, 0)),
            pl.BlockSpec((1, _D), lambda i: (0, 0)),
            pl.BlockSpec((_D, _D), lambda i: (0, 0)),
            pl.BlockSpec((_R, _D), lambda i: (i, 0)),
        ],
        out_specs=[pl.BlockSpec((_R, _D), lambda i: (i, 0))] * 2,
        out_shape=[jax.ShapeDtypeStruct((_N, _D), jnp.float32)] * 2,
    )(s, ri, b, w, ro)


def _tc_final_body(s_ref, ri_ref, b_ref, y_ref):
    y_ref[...] = jnp.maximum(
        (s_ref[0] + s_ref[1]) * ri_ref[...] + b_ref[...], 0.0
    )


def _final(s, ri, b):
    return pl.pallas_call(
        _tc_final_body,
        grid=(_G,),
        in_specs=[
            pl.BlockSpec((_NC, _R, _D), lambda i: (0, i, 0)),
            pl.BlockSpec((_R, _D), lambda i: (i, 0)),
            pl.BlockSpec((1, _D), lambda i: (0, 0)),
        ],
        out_specs=pl.BlockSpec((_R, _D), lambda i: (i, 0)),
        out_shape=jax.ShapeDtypeStruct((_N, _D), jnp.float32),
    )(s, ri, b)


@jax.jit
def kernel(data, edge_index, W1, b1, W2, b2, W3, b3):
    srcg = edge_index[0].reshape(_NW, _NCHUNK, _CH)
    dstg = edge_index[1].reshape(_NW, _NCHUNK, _CH)

    dp = _sc_degrees(srcg, dstg)
    ro, ri = _degprep(dp)

    h1 = _prep(data, W1, ro)
    s1 = _sc_gather_scatter(h1, srcg, dstg)
    y1, h2 = _step(s1, ri, b1.reshape(1, _D), W2, ro)
    s2 = _sc_gather_scatter(h2, srcg, dstg)
    y2, h3 = _step(s2, ri, b2.reshape(1, _D), W3, ro)
    s3 = _sc_gather_scatter(h3, srcg, dstg)
    y3 = _final(s3, ri, b3.reshape(1, _D))

    return (y3, jnp.stack([y1, y2, y3]))


# speedup vs baseline: 11.1701x; 11.1701x over previous
import functools

import jax
import jax.numpy as jnp
from jax import lax
from jax.experimental import pallas as pl
from jax.experimental.pallas import tpu as pltpu
from jax.experimental.pallas import tpu_sc as plsc

_N = 10000
_E = 320000
_D = 128
_NP = 10240
_PADNODE = _NP - 1
_NC = 2
_NS = 16
_NW = _NC * _NS
_CH = 64
_NCHUNK = 160
_EP = _NW * _NCHUNK * _CH
_EPT = _EP // _NW
_RPT = _NP // _NS
_ZR = 128
_NZ = _RPT // _ZR
_NZC = _RPT // _CH

_mesh = plsc.VectorSubcoreMesh(core_axis_name="c", subcore_axis_name="s")


@functools.partial(
    pl.kernel,
    out_type=jax.ShapeDtypeStruct((_NC, _NP, _D), jnp.float32),
    mesh=_mesh,
    scratch_types=[
        pltpu.VMEM((_NCHUNK, _CH), jnp.int32),
        pltpu.VMEM((_CH, _D), jnp.float32),
        pltpu.VMEM_SHARED((_NP, _D), jnp.float32),
    ],
)
def _sc_degcount(idxg, out, idx_v, ones_v, acc):
    cid = lax.axis_index("c")
    sid = lax.axis_index("s")
    wid = sid * _NC + cid
    pltpu.sync_copy(idxg.at[wid], idx_v)
    zero16 = jnp.zeros((16,), jnp.float32)
    one16 = jnp.ones((16,), jnp.float32)

    def zfill(j, c):
        for k in range(_D // 16):
            ones_v[j, pl.ds(k * 16, 16)] = zero16
        return c

    lax.fori_loop(0, _CH, zfill, 0)
    base = sid * _RPT
    for t in range(_NZC):
        pltpu.sync_copy(ones_v, acc.at[pl.ds(base + t * _CH, _CH)])

    def fill(j, c):
        ones_v[j, pl.ds(0, 16)] = one16
        return c

    lax.fori_loop(0, _CH, fill, 0)
    plsc.subcore_barrier()

    def body(j, c):
        pltpu.sync_copy(ones_v, acc.at[idx_v.at[j]], add=True)
        return c

    lax.fori_loop(0, _NCHUNK, body, 0)
    plsc.subcore_barrier()
    for t in range(_NZ):
        off = base + t * _ZR
        pltpu.sync_copy(acc.at[pl.ds(off, _ZR)], out.at[cid, pl.ds(off, _ZR)])


@functools.partial(
    pl.kernel,
    out_type=jax.ShapeDtypeStruct((_NC, _NP, _D), jnp.float32),
    mesh=_mesh,
    scratch_types=[
        pltpu.VMEM((_NCHUNK // 2, _CH), jnp.int32),
        pltpu.VMEM((_NCHUNK // 2, _CH), jnp.int32),
        pltpu.VMEM((_CH, _D), jnp.float32),
        pltpu.VMEM((_CH, _D), jnp.float32),
        pltpu.VMEM_SHARED((_NP, _D), jnp.float32),
        pltpu.SemaphoreType.DMA,
        pltpu.SemaphoreType.DMA,
    ],
)
def _sc_gather_scatter(h, srcg, dstg, out, src_v, dst_v, rows_a, rows_b, accum, sem_a, sem_b):
    cid = lax.axis_index("c")
    sid = lax.axis_index("s")
    wid = sid * _NC + cid
    zero16 = jnp.zeros((16,), jnp.float32)

    def fill_zero(j, c):
        for k in range(_D // 16):
            rows_a[j, pl.ds(k * 16, 16)] = zero16
        return c

    lax.fori_loop(0, _CH, fill_zero, 0)
    base = sid * _RPT
    for t in range(_NZC):
        pltpu.sync_copy(rows_a, accum.at[pl.ds(base + t * _CH, _CH)])
    plsc.subcore_barrier()

    _HC = _NCHUNK // 2
    _NPAIR = _HC // 2
    for half in range(2):
        pltpu.sync_copy(srcg.at[wid, pl.ds(half * _HC, _HC)], src_v)
        pltpu.sync_copy(dstg.at[wid, pl.ds(half * _HC, _HC)], dst_v)
        pltpu.async_copy(h.at[src_v.at[0]], rows_a, sem_a)

        def body(i, c):
            ja = 2 * i
            jb = 2 * i + 1
            pltpu.async_copy(h.at[src_v.at[jb]], rows_b, sem_b)
            pltpu.make_async_copy(h.at[src_v.at[ja]], rows_a, sem_a).wait()
            pltpu.sync_copy(rows_a, accum.at[dst_v.at[ja]], add=True)

            @pl.when(i < _NPAIR - 1)
            def _():
                pltpu.async_copy(h.at[src_v.at[ja + 2]], rows_a, sem_a)

            pltpu.make_async_copy(h.at[src_v.at[jb]], rows_b, sem_b).wait()
            pltpu.sync_copy(rows_b, accum.at[dst_v.at[jb]], add=True)
            return c

        lax.fori_loop(0, _NPAIR, body, 0)
    plsc.subcore_barrier()
    for t in range(_NZ):
        off = base + t * _ZR
        pltpu.sync_copy(accum.at[pl.ds(off, _ZR)], out.at[cid, pl.ds(off, _ZR)])


@functools.partial(
    pl.kernel,
    out_type=jax.ShapeDtypeStruct((_NW, _EPT, _D), jnp.float32),
    mesh=_mesh,
    scratch_types=[
        pltpu.VMEM((_NCHUNK // 2, _CH), jnp.int32),
        pltpu.VMEM((_CH, _D), jnp.float32),
        pltpu.VMEM((_CH, _D), jnp.float32),
        pltpu.VMEM_SHARED((_NP, _D), jnp.float32),
        pltpu.SemaphoreType.DMA,
        pltpu.SemaphoreType.DMA,
        pltpu.SemaphoreType.DMA,
        pltpu.SemaphoreType.DMA,
    ],
)
def _sc_gather_spill(h, srcg, msg, src_v, rows_a, rows_b, table, g_a, g_b, w_a, w_b):
    cid = lax.axis_index("c")
    sid = lax.axis_index("s")
    wid = sid * _NC + cid
    base = sid * _RPT
    pltpu.sync_copy(h.at[pl.ds(base, _RPT)], table.at[pl.ds(base, _RPT)])
    plsc.subcore_barrier()

    _HC = _NCHUNK // 2
    _NPAIR = _HC // 2
    for half in range(2):
        hb = half * _HC
        pltpu.sync_copy(srcg.at[wid, pl.ds(hb, _HC)], src_v)
        pltpu.async_copy(table.at[src_v.at[0]], rows_a, g_a)

        def body(i, c):
            ja = 2 * i
            jb = 2 * i + 1
            pltpu.make_async_copy(table.at[src_v.at[ja]], rows_a, g_a).wait()
            pltpu.async_copy(
                rows_a, msg.at[wid, pl.ds((hb + ja) * _CH, _CH)], w_a
            )

            @pl.when(i > 0)
            def _():
                pltpu.make_async_copy(
                    rows_b, msg.at[wid, pl.ds((hb + jb - 2) * _CH, _CH)], w_b
                ).wait()

            pltpu.async_copy(table.at[src_v.at[jb]], rows_b, g_b)
            pltpu.make_async_copy(table.at[src_v.at[jb]], rows_b, g_b).wait()
            pltpu.async_copy(
                rows_b, msg.at[wid, pl.ds((hb + jb) * _CH, _CH)], w_b
            )

            @pl.when(i < _NPAIR - 1)
            def _():
                pltpu.make_async_copy(
                    rows_a, msg.at[wid, pl.ds((hb + ja) * _CH, _CH)], w_a
                ).wait()
                pltpu.async_copy(table.at[src_v.at[ja + 2]], rows_a, g_a)

            return c

        lax.fori_loop(0, _NPAIR, body, 0)
        pltpu.make_async_copy(
            rows_a, msg.at[wid, pl.ds((hb + _HC - 2) * _CH, _CH)], w_a
        ).wait()
        pltpu.make_async_copy(
            rows_b, msg.at[wid, pl.ds((hb + _HC - 1) * _CH, _CH)], w_b
        ).wait()


@functools.partial(
    pl.kernel,
    out_type=jax.ShapeDtypeStruct((_NC, _NP, _D), jnp.float32),
    mesh=_mesh,
    scratch_types=[
        pltpu.VMEM((_NCHUNK // 2, _CH), jnp.int32),
        pltpu.VMEM((_CH, _D), jnp.float32),
        pltpu.VMEM((_CH, _D), jnp.float32),
        pltpu.VMEM_SHARED((_NP, _D), jnp.float32),
        pltpu.SemaphoreType.DMA,
        pltpu.SemaphoreType.DMA,
    ],
)
def _sc_spill_scatter(msg, dstg, out, dst_v, rows_a, rows_b, accum, r_a, r_b):
    cid = lax.axis_index("c")
    sid = lax.axis_index("s")
    wid = sid * _NC + cid
    zero16 = jnp.zeros((16,), jnp.float32)

    def fill_zero(j, c):
        for k in range(_D // 16):
            rows_a[j, pl.ds(k * 16, 16)] = zero16
        return c

    lax.fori_loop(0, _CH, fill_zero, 0)
    base = sid * _RPT
    for t in range(_NZC):
        pltpu.sync_copy(rows_a, accum.at[pl.ds(base + t * _CH, _CH)])
    plsc.subcore_barrier()

    _HC = _NCHUNK // 2
    _NPAIR = _HC // 2
    for half in range(2):
        hb = half * _HC
        pltpu.sync_copy(dstg.at[wid, pl.ds(hb, _HC)], dst_v)
        pltpu.async_copy(msg.at[wid, pl.ds(hb * _CH, _CH)], rows_a, r_a)

        def body(i, c):
            ja = 2 * i
            jb = 2 * i + 1
            pltpu.async_copy(
                msg.at[wid, pl.ds((hb + jb) * _CH, _CH)], rows_b, r_b
            )
            pltpu.make_async_copy(
                msg.at[wid, pl.ds((hb + ja) * _CH, _CH)], rows_a, r_a
            ).wait()
            pltpu.sync_copy(rows_a, accum.at[dst_v.at[ja]], add=True)

            @pl.when(i < _NPAIR - 1)
            def _():
                pltpu.async_copy(
                    msg.at[wid, pl.ds((hb + ja + 2) * _CH, _CH)], rows_a, r_a
                )

            pltpu.make_async_copy(
                msg.at[wid, pl.ds((hb + jb) * _CH, _CH)], rows_b, r_b
            ).wait()
            pltpu.sync_copy(rows_b, accum.at[dst_v.at[jb]], add=True)
            return c

        lax.fori_loop(0, _NPAIR, body, 0)
    plsc.subcore_barrier()
    for t in range(_NZ):
        off = base + t * _ZR
        pltpu.sync_copy(accum.at[pl.ds(off, _ZR)], out.at[cid, pl.ds(off, _ZR)])


_R = 2048
_G = _NP // _R


def _tc_degprep_body(dpo_ref, dpi_ref, ro_ref, ri_ref):
    do = dpo_ref[0] + dpo_ref[1]
    di = dpi_ref[0] + dpi_ref[1]
    dego = jnp.sum(do[:, 0:16], axis=1) * (1.0 / 16.0)
    degi = jnp.sum(di[:, 0:16], axis=1) * (1.0 / 16.0)
    ro = lax.rsqrt(jnp.maximum(dego, 1.0))
    ri = lax.rsqrt(jnp.maximum(degi, 1.0))
    ro_ref[...] = jnp.broadcast_to(ro[:, None], (_R, _D))
    ri_ref[...] = jnp.broadcast_to(ri[:, None], (_R, _D))


def _degprep(dp_o, dp_i):
    return pl.pallas_call(
        _tc_degprep_body,
        grid=(_G,),
        in_specs=[
            pl.BlockSpec((_NC, _R, _D), lambda i: (0, i, 0)),
            pl.BlockSpec((_NC, _R, _D), lambda i: (0, i, 0)),
        ],
        out_specs=[pl.BlockSpec((_R, _D), lambda i: (i, 0))] * 2,
        out_shape=[jax.ShapeDtypeStruct((_NP, _D), jnp.float32)] * 2,
    )(dp_o, dp_i)


def _tc_prep_body(x_ref, w_ref, ro_ref, h_ref):
    h_ref[...] = (
        jnp.dot(x_ref[...], w_ref[...], preferred_element_type=jnp.float32)
        * ro_ref[...]
    )


def _prep(x, w, ro):
    return pl.pallas_call(
        _tc_prep_body,
        grid=(_G,),
        in_specs=[
            pl.BlockSpec((_R, _D), lambda i: (i, 0)),
            pl.BlockSpec((_D, _D), lambda i: (0, 0)),
            pl.BlockSpec((_R, _D), lambda i: (i, 0)),
        ],
        out_specs=pl.BlockSpec((_R, _D), lambda i: (i, 0)),
        out_shape=jax.ShapeDtypeStruct((_NP, _D), jnp.float32),
    )(x, w, ro)


def _tc_step_body(s_ref, ri_ref, b_ref, w_ref, ro_ref, y_ref, h_ref):
    agg = (s_ref[0] + s_ref[1]) * ri_ref[...] + b_ref[...]
    y = jnp.maximum(agg, 0.0)
    y_ref[...] = y
    h_ref[...] = (
        jnp.dot(y, w_ref[...], preferred_element_type=jnp.float32) * ro_ref[...]
    )


def _step(s, ri, b, w, ro):
    return pl.pallas_call(
        _tc_step_body,
        grid=(_G,),
        in_specs=[
            pl.BlockSpec((_NC, _R, _D), lambda i: (0, i, 0)),
            pl.BlockSpec((_R, _D), lambda i: (i, 0)),
            pl.BlockSpec((1, _D), lambda i: (0, 0)),
            pl.BlockSpec((_D, _D), lambda i: (0, 0)),
            pl.BlockSpec((_R, _D), lambda i: (i, 0)),
        ],
        out_specs=[pl.BlockSpec((_R, _D), lambda i: (i, 0))] * 2,
        out_shape=[jax.ShapeDtypeStruct((_NP, _D), jnp.float32)] * 2,
    )(s, ri, b, w, ro)


def _tc_final_body(s_ref, ri_ref, b_ref, y_ref):
    y_ref[...] = jnp.maximum(
        (s_ref[0] + s_ref[1]) * ri_ref[...] + b_ref[...], 0.0
    )


def _final(s, ri, b):
    return pl.pallas_call(
        _tc_final_body,
        grid=(_G,),
        in_specs=[
            pl.BlockSpec((_NC, _R, _D), lambda i: (0, i, 0)),
            pl.BlockSpec((_R, _D), lambda i: (i, 0)),
            pl.BlockSpec((1, _D), lambda i: (0, 0)),
        ],
        out_specs=pl.BlockSpec((_R, _D), lambda i: (i, 0)),
        out_shape=jax.ShapeDtypeStruct((_NP, _D), jnp.float32),
    )(s, ri, b)


@jax.jit
def kernel(data, edge_index, W1, b1, W2, b2, W3, b3):
    pad = jnp.full((_EP - _E,), _PADNODE, dtype=jnp.int32)
    src = jnp.concatenate([edge_index[0], pad])
    dst = jnp.concatenate([edge_index[1], pad])
    srcg = src.reshape(_NW, _NCHUNK, _CH)
    dstg = dst.reshape(_NW, _NCHUNK, _CH)
    xp = jnp.pad(data, ((0, _NP - _N), (0, 0)))

    dp_o = _sc_degcount(srcg)
    dp_i = _sc_degcount(dstg)
    ro, ri = _degprep(dp_o, dp_i)

    def _sc_layer(h):
        msg = _sc_gather_spill(h, srcg)
        return _sc_spill_scatter(msg, dstg)

    h1 = _prep(xp, W1, ro)
    s1 = _sc_layer(h1)
    y1, h2 = _step(s1, ri, b1.reshape(1, _D), W2, ro)
    s2 = _sc_layer(h2)
    y2, h3 = _step(s2, ri, b2.reshape(1, _D), W3, ro)
    s3 = _sc_layer(h3)
    y3 = _final(s3, ri, b3.reshape(1, _D))

    return (y3[:_N], jnp.stack([y1[:_N], y2[:_N], y3[:_N]]))


# speedup vs baseline: 12.8060x; 1.1465x over previous
import functools

import jax
import jax.numpy as jnp
from jax import lax
from jax.experimental import pallas as pl
from jax.experimental.pallas import tpu as pltpu
from jax.experimental.pallas import tpu_sc as plsc

_N = 10000
_E = 320000
_D = 128
_NP = 10240
_PADNODE = _NP - 1
_NC = 2
_NS = 16
_NW = _NC * _NS
_CH = 64
_NCHUNK = 160
_EP = _NW * _NCHUNK * _CH
_EPT = _EP // _NW
_RPT = _NP // _NS
_ZR = 128
_NZ = _RPT // _ZR
_NZC = _RPT // _CH

_mesh = plsc.VectorSubcoreMesh(core_axis_name="c", subcore_axis_name="s")


@functools.partial(
    pl.kernel,
    out_type=jax.ShapeDtypeStruct((_NC, _NP, _D), jnp.float32),
    mesh=_mesh,
    scratch_types=[
        pltpu.VMEM((_NCHUNK // 2, _CH), jnp.int32),
        pltpu.VMEM((_NCHUNK // 2, _CH), jnp.int32),
        pltpu.VMEM((_CH, _D), jnp.float32),
        pltpu.VMEM((_CH, _D), jnp.float32),
        pltpu.VMEM_SHARED((_NP, _D), jnp.float32),
    ],
)
def _sc_degcount(srcg, dstg, out, src_v, dst_v, ones_s, ones_d, acc):
    cid = lax.axis_index("c")
    sid = lax.axis_index("s")
    wid = sid * _NC + cid
    zero16 = jnp.zeros((16,), jnp.float32)
    one16 = jnp.ones((16,), jnp.float32)

    def zfill(j, c):
        for k in range(_D // 16):
            ones_s[j, pl.ds(k * 16, 16)] = zero16
            ones_d[j, pl.ds(k * 16, 16)] = zero16
        return c

    lax.fori_loop(0, _CH, zfill, 0)
    base = sid * _RPT
    for t in range(_NZC):
        pltpu.sync_copy(ones_s, acc.at[pl.ds(base + t * _CH, _CH)])

    def fill(j, c):
        ones_s[j, pl.ds(0, 16)] = one16
        ones_d[j, pl.ds(16, 16)] = one16
        return c

    lax.fori_loop(0, _CH, fill, 0)
    plsc.subcore_barrier()

    _HC = _NCHUNK // 2
    for half in range(2):
        pltpu.sync_copy(srcg.at[wid, pl.ds(half * _HC, _HC)], src_v)
        pltpu.sync_copy(dstg.at[wid, pl.ds(half * _HC, _HC)], dst_v)

        def body(j, c):
            pltpu.sync_copy(ones_s, acc.at[src_v.at[j]], add=True)
            pltpu.sync_copy(ones_d, acc.at[dst_v.at[j]], add=True)
            return c

        lax.fori_loop(0, _HC, body, 0)
    plsc.subcore_barrier()
    for t in range(_NZ):
        off = base + t * _ZR
        pltpu.sync_copy(acc.at[pl.ds(off, _ZR)], out.at[cid, pl.ds(off, _ZR)])


@functools.partial(
    pl.kernel,
    out_type=jax.ShapeDtypeStruct((_NW, _EPT, _D), jnp.float32),
    mesh=_mesh,
    scratch_types=[
        pltpu.VMEM((_NCHUNK // 2, _CH), jnp.int32),
        pltpu.VMEM((_CH, _D), jnp.float32),
        pltpu.VMEM((_CH, _D), jnp.float32),
        pltpu.VMEM_SHARED((_NP, _D), jnp.float32),
        pltpu.SemaphoreType.DMA,
        pltpu.SemaphoreType.DMA,
        pltpu.SemaphoreType.DMA,
        pltpu.SemaphoreType.DMA,
    ],
)
def _sc_gather_spill(h, srcg, msg, src_v, rows_a, rows_b, table, g_a, g_b, w_a, w_b):
    cid = lax.axis_index("c")
    sid = lax.axis_index("s")
    wid = sid * _NC + cid
    base = sid * _RPT
    pltpu.sync_copy(h.at[pl.ds(base, _RPT)], table.at[pl.ds(base, _RPT)])
    plsc.subcore_barrier()

    _HC = _NCHUNK // 2
    _NPAIR = _HC // 2
    for half in range(2):
        hb = half * _HC
        pltpu.sync_copy(srcg.at[wid, pl.ds(hb, _HC)], src_v)
        pltpu.async_copy(table.at[src_v.at[0]], rows_a, g_a)

        def body(i, c):
            ja = 2 * i
            jb = 2 * i + 1
            pltpu.make_async_copy(table.at[src_v.at[ja]], rows_a, g_a).wait()
            pltpu.async_copy(
                rows_a, msg.at[wid, pl.ds((hb + ja) * _CH, _CH)], w_a
            )

            @pl.when(i > 0)
            def _():
                pltpu.make_async_copy(
                    rows_b, msg.at[wid, pl.ds((hb + jb - 2) * _CH, _CH)], w_b
                ).wait()

            pltpu.async_copy(table.at[src_v.at[jb]], rows_b, g_b)
            pltpu.make_async_copy(table.at[src_v.at[jb]], rows_b, g_b).wait()
            pltpu.async_copy(
                rows_b, msg.at[wid, pl.ds((hb + jb) * _CH, _CH)], w_b
            )

            @pl.when(i < _NPAIR - 1)
            def _():
                pltpu.make_async_copy(
                    rows_a, msg.at[wid, pl.ds((hb + ja) * _CH, _CH)], w_a
                ).wait()
                pltpu.async_copy(table.at[src_v.at[ja + 2]], rows_a, g_a)

            return c

        lax.fori_loop(0, _NPAIR, body, 0)
        pltpu.make_async_copy(
            rows_a, msg.at[wid, pl.ds((hb + _HC - 2) * _CH, _CH)], w_a
        ).wait()
        pltpu.make_async_copy(
            rows_b, msg.at[wid, pl.ds((hb + _HC - 1) * _CH, _CH)], w_b
        ).wait()


@functools.partial(
    pl.kernel,
    out_type=jax.ShapeDtypeStruct((_NC, _NP, _D), jnp.float32),
    mesh=_mesh,
    scratch_types=[
        pltpu.VMEM((_NCHUNK // 2, _CH), jnp.int32),
        pltpu.VMEM((_CH, _D), jnp.float32),
        pltpu.VMEM((_CH, _D), jnp.float32),
        pltpu.VMEM((_CH, _D), jnp.float32),
        pltpu.VMEM((_CH, _D), jnp.float32),
        pltpu.VMEM_SHARED((_NP, _D), jnp.float32),
        pltpu.SemaphoreType.DMA,
        pltpu.SemaphoreType.DMA,
        pltpu.SemaphoreType.DMA,
        pltpu.SemaphoreType.DMA,
    ],
)
def _sc_spill_scatter(
    msg, dstg, out, dst_v, rows_a, rows_b, rows_c, rows_d, accum, r_a, r_b, r_c, r_d
):
    cid = lax.axis_index("c")
    sid = lax.axis_index("s")
    wid = sid * _NC + cid
    zero16 = jnp.zeros((16,), jnp.float32)

    def fill_zero(j, c):
        for k in range(_D // 16):
            rows_a[j, pl.ds(k * 16, 16)] = zero16
        return c

    lax.fori_loop(0, _CH, fill_zero, 0)
    base = sid * _RPT
    for t in range(_NZC):
        pltpu.sync_copy(rows_a, accum.at[pl.ds(base + t * _CH, _CH)])
    plsc.subcore_barrier()

    bufs = (rows_a, rows_b, rows_c, rows_d)
    sems = (r_a, r_b, r_c, r_d)
    _HC = _NCHUNK // 2
    _NG = _HC // 4
    for half in range(2):
        hb = half * _HC
        pltpu.sync_copy(dstg.at[wid, pl.ds(hb, _HC)], dst_v)
        for q in range(4):
            pltpu.async_copy(
                msg.at[wid, pl.ds((hb + q) * _CH, _CH)], bufs[q], sems[q]
            )

        def body(i, c):
            for q in range(4):
                j = 4 * i + q
                pltpu.make_async_copy(
                    msg.at[wid, pl.ds((hb + j) * _CH, _CH)], bufs[q], sems[q]
                ).wait()
                pltpu.sync_copy(bufs[q], accum.at[dst_v.at[j]], add=True)

                @pl.when(i < _NG - 1)
                def _():
                    pltpu.async_copy(
                        msg.at[wid, pl.ds((hb + j + 4) * _CH, _CH)],
                        bufs[q],
                        sems[q],
                    )

            return c

        lax.fori_loop(0, _NG, body, 0)
    plsc.subcore_barrier()
    for t in range(_NZ):
        off = base + t * _ZR
        pltpu.sync_copy(accum.at[pl.ds(off, _ZR)], out.at[cid, pl.ds(off, _ZR)])


_R = 2048
_G = _NP // _R


def _tc_degprep_body(dp_ref, m_ref, ro_ref, ri_ref, h_ref):
    d = dp_ref[0] + dp_ref[1]
    dego = jnp.sum(d[:, 0:16], axis=1) * (1.0 / 16.0)
    degi = jnp.sum(d[:, 16:32], axis=1) * (1.0 / 16.0)
    ro = lax.rsqrt(jnp.maximum(dego, 1.0))
    ri = lax.rsqrt(jnp.maximum(degi, 1.0))
    rob = jnp.broadcast_to(ro[:, None], (_R, _D))
    ro_ref[...] = rob
    ri_ref[...] = jnp.broadcast_to(ri[:, None], (_R, _D))
    h_ref[...] = m_ref[...] * rob


def _degprep(dp, m1):
    return pl.pallas_call(
        _tc_degprep_body,
        grid=(_G,),
        in_specs=[
            pl.BlockSpec((_NC, _R, _D), lambda i: (0, i, 0)),
            pl.BlockSpec((_R, _D), lambda i: (i, 0)),
        ],
        out_specs=[pl.BlockSpec((_R, _D), lambda i: (i, 0))] * 3,
        out_shape=[jax.ShapeDtypeStruct((_NP, _D), jnp.float32)] * 3,
    )(dp, m1)


def _tc_matmul_body(x_ref, w_ref, h_ref):
    h_ref[...] = jnp.dot(
        x_ref[...], w_ref[...], preferred_element_type=jnp.float32
    )


def _matmul(x, w):
    return pl.pallas_call(
        _tc_matmul_body,
        grid=(_G,),
        in_specs=[
            pl.BlockSpec((_R, _D), lambda i: (i, 0)),
            pl.BlockSpec((_D, _D), lambda i: (0, 0)),
        ],
        out_specs=pl.BlockSpec((_R, _D), lambda i: (i, 0)),
        out_shape=jax.ShapeDtypeStruct((_NP, _D), jnp.float32),
    )(x, w)


def _tc_step_body(s_ref, ri_ref, b_ref, w_ref, ro_ref, y_ref, h_ref):
    agg = (s_ref[0] + s_ref[1]) * ri_ref[...] + b_ref[...]
    y = jnp.maximum(agg, 0.0)
    y_ref[...] = y
    h_ref[...] = (
        jnp.dot(y, w_ref[...], preferred_element_type=jnp.float32) * ro_ref[...]
    )


def _step(s, ri, b, w, ro):
    return pl.pallas_call(
        _tc_step_body,
        grid=(_G,),
        in_specs=[
            pl.BlockSpec((_NC, _R, _D), lambda i: (0, i, 0)),
            pl.BlockSpec((_R, _D), lambda i: (i, 0)),
            pl.BlockSpec((1, _D), lambda i: (0, 0)),
            pl.BlockSpec((_D, _D), lambda i: (0, 0)),
            pl.BlockSpec((_R, _D), lambda i: (i, 0)),
        ],
        out_specs=[pl.BlockSpec((_R, _D), lambda i: (i, 0))] * 2,
        out_shape=[jax.ShapeDtypeStruct((_NP, _D), jnp.float32)] * 2,
    )(s, ri, b, w, ro)


def _tc_final_body(s_ref, ri_ref, b_ref, y_ref):
    y_ref[...] = jnp.maximum(
        (s_ref[0] + s_ref[1]) * ri_ref[...] + b_ref[...], 0.0
    )


def _final(s, ri, b):
    return pl.pallas_call(
        _tc_final_body,
        grid=(_G,),
        in_specs=[
            pl.BlockSpec((_NC, _R, _D), lambda i: (0, i, 0)),
            pl.BlockSpec((_R, _D), lambda i: (i, 0)),
            pl.BlockSpec((1, _D), lambda i: (0, 0)),
        ],
        out_specs=pl.BlockSpec((_R, _D), lambda i: (i, 0)),
        out_shape=jax.ShapeDtypeStruct((_NP, _D), jnp.float32),
    )(s, ri, b)


@jax.jit
def kernel(data, edge_index, W1, b1, W2, b2, W3, b3):
    pad = jnp.full((_EP - _E,), _PADNODE, dtype=jnp.int32)
    src = jnp.concatenate([edge_index[0], pad])
    dst = jnp.concatenate([edge_index[1], pad])
    srcg = src.reshape(_NW, _NCHUNK, _CH)
    dstg = dst.reshape(_NW, _NCHUNK, _CH)
    xp = jnp.pad(data, ((0, _NP - _N), (0, 0)))

    m1 = _matmul(xp, W1)
    dp = _sc_degcount(srcg, dstg)
    ro, ri, h1 = _degprep(dp, m1)

    def _sc_layer(h):
        msg = _sc_gather_spill(h, srcg)
        return _sc_spill_scatter(msg, dstg)

    s1 = _sc_layer(h1)
    y1, h2 = _step(s1, ri, b1.reshape(1, _D), W2, ro)
    s2 = _sc_layer(h2)
    y2, h3 = _step(s2, ri, b2.reshape(1, _D), W3, ro)
    s3 = _sc_layer(h3)
    y3 = _final(s3, ri, b3.reshape(1, _D))

    return (y3[:_N], jnp.stack([y1[:_N], y2[:_N], y3[:_N]]))
